# Initial kernel scaffold; baseline (speedup 1.0000x reference)
import jax
import jax.numpy as jnp
from jax.experimental import pallas as pl


def kernel(node_feats, edge_index, W_self1, W_neigh1, b1, W_self2, W_neigh2, b2, W_res, W_mlp, b_mlp):
    raise NotImplementedError("write your pallas kernel here")


# speedup vs baseline: 4.8287x; 4.8287x over previous
import jax
import jax.numpy as jnp
from jax import lax
from jax.experimental import pallas as pl
from jax.experimental.pallas import tpu as pltpu
from jax.experimental.pallas import tpu_sc as plsc

D = 128
DH = 64
NC = 2
NS = 16
CHUNK = 128


def _make_seg_kernel(nch, n_rows, rows_per_tile, with_deg):
    mesh = plsc.VectorSubcoreMesh(
        core_axis_name="c", subcore_axis_name="s", num_cores=NC, num_subcores=NS
    )
    out_type = [jax.ShapeDtypeStruct((NC, n_rows, DH), jnp.float32)]
    if with_deg:
        out_type.append(jax.ShapeDtypeStruct((n_rows, 16), jnp.float32))
    scratch_types = [
        pltpu.VMEM((nch, CHUNK), jnp.int32),
        pltpu.VMEM((nch, CHUNK), jnp.int32),
        pltpu.VMEM((CHUNK, DH), jnp.float32),
        pltpu.VMEM((CHUNK, DH), jnp.float32),
        pltpu.VMEM((CHUNK, 16), jnp.float32),
        pltpu.VMEM((CHUNK, 16), jnp.float32),
        pltpu.VMEM_SHARED((n_rows, DH), jnp.float32),
        pltpu.VMEM_SHARED((n_rows, 16), jnp.float32),
        pltpu.SemaphoreType.DMA,
        pltpu.SemaphoreType.DMA,
    ]

    def body(table, src_hbm, dst_hbm, *rest):
        if with_deg:
            (agg_out, deg_out, src_v, dst_v, buf_a, buf_b, ones_v, z16,
             agg_sh, deg_sh, sem_a, sem_b) = rest
        else:
            (agg_out, src_v, dst_v, buf_a, buf_b, ones_v, z16,
             agg_sh, deg_sh, sem_a, sem_b) = rest
        c = lax.axis_index("c")
        s = lax.axis_index("s")
        base = s * rows_per_tile
        my_tab = table.at[c]

        pltpu.sync_copy(src_hbm.at[pl.ds(s * nch, nch)], src_v)
        pltpu.sync_copy(dst_hbm.at[pl.ds(s * nch, nch)], dst_v)

        zv = jnp.zeros((16,), jnp.float32)

        @pl.loop(0, CHUNK * (DH // 16))
        def _(t):
            i = t // (DH // 16)
            k = t % (DH // 16)
            buf_a[i, pl.ds(k * 16, 16)] = zv

        @pl.loop(0, CHUNK)
        def _(i):
            z16[i, pl.ds(0, 16)] = zv
            ones_v[i, pl.ds(0, 16)] = zv + 1.0

        for r in range(rows_per_tile // CHUNK):
            rb = base + r * CHUNK
            pltpu.sync_copy(buf_a, agg_sh.at[pl.ds(rb, CHUNK)])
            if with_deg:
                @pl.when(c == 0)
                def _():
                    pltpu.sync_copy(z16, deg_sh.at[pl.ds(rb, CHUNK)])
        plsc.subcore_barrier()

        def issue(j, buf, sem):
            pltpu.async_copy(my_tab.at[src_v.at[j]], buf, sem)

        def wait(buf, sem):
            pltpu.make_async_copy(my_tab.at[src_v.at[0]], buf, sem).wait()

        def scat(j, buf):
            pltpu.sync_copy(buf, agg_sh.at[dst_v.at[j]], add=True)
            if with_deg:
                @pl.when(c == 0)
                def _():
                    pltpu.sync_copy(ones_v, deg_sh.at[dst_v.at[j]], add=True)

        issue(0, buf_a, sem_a)

        @pl.loop(0, nch // 2 - 1)
        def _(t):
            j0 = 2 * t
            issue(j0 + 1, buf_b, sem_b)
            wait(buf_a, sem_a)
            scat(j0, buf_a)
            issue(j0 + 2, buf_a, sem_a)
            wait(buf_b, sem_b)
            scat(j0 + 1, buf_b)

        issue(nch - 1, buf_b, sem_b)
        wait(buf_a, sem_a)
        scat(nch - 2, buf_a)
        wait(buf_b, sem_b)
        scat(nch - 1, buf_b)

        plsc.subcore_barrier()

        for r in range(rows_per_tile // CHUNK):
            rb = base + r * CHUNK
            pltpu.sync_copy(agg_sh.at[pl.ds(rb, CHUNK)], buf_a)
            pltpu.sync_copy(buf_a, agg_out.at[c, pl.ds(rb, CHUNK)])
            if with_deg:
                @pl.when(c == 0)
                def _():
                    pltpu.sync_copy(deg_sh.at[pl.ds(rb, CHUNK)], z16)
                    pltpu.sync_copy(z16, deg_out.at[pl.ds(rb, CHUNK)])

    return pl.kernel(
        body, out_type=out_type, mesh=mesh, scratch_types=scratch_types,
        compiler_params=pltpu.CompilerParams(use_tc_tiling_on_sc=False),
    )


def _sage_layer1(x2, aggp, deg, w_self, w_neigh, b):
    n_rows = x2.shape[1]
    blk = 1024

    def body(x_ref, a_ref, d_ref, ws_ref, wn_ref, b_ref, o_ref):
        x = jnp.concatenate([x_ref[0], x_ref[1]], axis=1)
        agg = jnp.concatenate([a_ref[0], a_ref[1]], axis=1)
        mean = agg / jnp.maximum(d_ref[:, 0:1], 1.0)
        h = jnp.dot(x, ws_ref[...], preferred_element_type=jnp.float32)
        h = h + jnp.dot(mean, wn_ref[...], preferred_element_type=jnp.float32)
        h = h + b_ref[...]
        h = jnp.maximum(h, 0.0)
        o_ref[0] = h[:, :DH]
        o_ref[1] = h[:, DH:]

    return pl.pallas_call(
        body,
        grid=(n_rows // blk,),
        in_specs=[
            pl.BlockSpec((NC, blk, DH), lambda i: (0, i, 0)),
            pl.BlockSpec((NC, blk, DH), lambda i: (0, i, 0)),
            pl.BlockSpec((blk, 16), lambda i: (i, 0)),
            pl.BlockSpec((D, D), lambda i: (0, 0)),
            pl.BlockSpec((D, D), lambda i: (0, 0)),
            pl.BlockSpec((1, D), lambda i: (0, 0)),
        ],
        out_specs=pl.BlockSpec((NC, blk, DH), lambda i: (0, i, 0)),
        out_shape=jax.ShapeDtypeStruct((NC, n_rows, DH), jnp.float32),
    )(x2, aggp, deg, w_self, w_neigh, b.reshape(1, D))


def _sage_layer2(h2, aggp, deg, x2, w_self, w_neigh, b, w_res, w_mlp_pad, b_mlp_pad):
    n_rows = h2.shape[1]
    blk = 1024

    def body(h_ref, a_ref, d_ref, x_ref, ws_ref, wn_ref, b_ref, wr_ref, wm_ref, bm_ref, o_ref):
        h1 = jnp.concatenate([h_ref[0], h_ref[1]], axis=1)
        x = jnp.concatenate([x_ref[0], x_ref[1]], axis=1)
        agg = jnp.concatenate([a_ref[0], a_ref[1]], axis=1)
        mean = agg / jnp.maximum(d_ref[:, 0:1], 1.0)
        out = jnp.dot(h1, ws_ref[...], preferred_element_type=jnp.float32)
        out = out + jnp.dot(mean, wn_ref[...], preferred_element_type=jnp.float32)
        out = out + jnp.dot(x, wr_ref[...], preferred_element_type=jnp.float32)
        out = out + b_ref[...]
        o_ref[...] = jnp.dot(out, wm_ref[...], preferred_element_type=jnp.float32) + bm_ref[...]

    return pl.pallas_call(
        body,
        grid=(n_rows // blk,),
        in_specs=[
            pl.BlockSpec((NC, blk, DH), lambda i: (0, i, 0)),
            pl.BlockSpec((NC, blk, DH), lambda i: (0, i, 0)),
            pl.BlockSpec((blk, 16), lambda i: (i, 0)),
            pl.BlockSpec((NC, blk, DH), lambda i: (0, i, 0)),
            pl.BlockSpec((D, D), lambda i: (0, 0)),
            pl.BlockSpec((D, D), lambda i: (0, 0)),
            pl.BlockSpec((1, D), lambda i: (0, 0)),
            pl.BlockSpec((D, D), lambda i: (0, 0)),
            pl.BlockSpec((D, D), lambda i: (0, 0)),
            pl.BlockSpec((1, D), lambda i: (0, 0)),
        ],
        out_specs=pl.BlockSpec((blk, D), lambda i: (i, 0)),
        out_shape=jax.ShapeDtypeStruct((n_rows, D), jnp.float32),
    )(h2, aggp, deg, x2, w_self, w_neigh, b.reshape(1, D), w_res, w_mlp_pad, b_mlp_pad)


def kernel(node_feats, edge_index, W_self1, W_neigh1, b1, W_self2, W_neigh2, b2, W_res, W_mlp, b_mlp):
    n = node_feats.shape[0]
    e = edge_index.shape[1]
    src = edge_index[0]
    dst = edge_index[1]

    nch = -(-e // (NS * CHUNK))
    nch = -(-nch // 8) * 8
    e_pad = NS * nch * CHUNK
    src_p = jnp.concatenate([src, jnp.zeros((e_pad - e,), jnp.int32)])
    dst_p = jnp.concatenate([dst, jnp.full((e_pad - e,), n, jnp.int32)])
    src_p = src_p.reshape(e_pad // CHUNK, CHUNK)
    dst_p = dst_p.reshape(e_pad // CHUNK, CHUNK)

    rows_per_tile = -(-(n + 1) // (NS * CHUNK)) * CHUNK
    n_rows = rows_per_tile * NS

    x_pad = jnp.zeros((n_rows, D), jnp.float32).at[:n].set(node_feats)
    x2 = jnp.stack([x_pad[:, :DH], x_pad[:, DH:]])

    seg1 = _make_seg_kernel(nch, n_rows, rows_per_tile, with_deg=True)
    aggp1, deg = seg1(x2, src_p, dst_p)

    h2 = _sage_layer1(x2, aggp1, deg, W_self1, W_neigh1, b1)

    seg2 = _make_seg_kernel(nch, n_rows, rows_per_tile, with_deg=False)
    (aggp2,) = seg2(h2, src_p, dst_p)

    w_mlp_pad = jnp.zeros((D, D), jnp.float32).at[:, : W_mlp.shape[1]].set(W_mlp)
    b_mlp_pad = jnp.zeros((1, D), jnp.float32).at[0, : W_mlp.shape[1]].set(b_mlp)

    out = _sage_layer2(h2, aggp2, deg, x2, W_self2, W_neigh2, b2, W_res, w_mlp_pad, b_mlp_pad)
    return out[:n, : W_mlp.shape[1]]


# speedup vs baseline: 5.0403x; 1.0438x over previous
import jax
import jax.numpy as jnp
from jax import lax
from jax.experimental import pallas as pl
from jax.experimental.pallas import tpu as pltpu
from jax.experimental.pallas import tpu_sc as plsc

D = 128
DH = 64
NC = 2
NS = 16
CHUNK = 128


def _make_seg_kernel(nch, n_rows, rows_per_tile, with_deg):
    mesh = plsc.VectorSubcoreMesh(
        core_axis_name="c", subcore_axis_name="s", num_cores=NC, num_subcores=NS
    )
    K = 2
    T = nch // K
    assert nch % K == 0 and T % 2 == 0 and T >= 4
    out_type = [jax.ShapeDtypeStruct((NC, n_rows, DH), jnp.float32)]
    if with_deg:
        out_type.append(jax.ShapeDtypeStruct((NC, n_rows, 16), jnp.float32))
    scratch_types = [
        pltpu.VMEM((nch, CHUNK), jnp.int32),
        pltpu.VMEM((nch, CHUNK), jnp.int32),
        pltpu.VMEM((K * CHUNK, DH), jnp.float32),
        pltpu.VMEM((K * CHUNK, DH), jnp.float32),
        pltpu.VMEM((CHUNK, 16), jnp.float32),
        pltpu.VMEM((CHUNK, 16), jnp.float32),
        pltpu.VMEM_SHARED((n_rows, DH), jnp.float32),
        pltpu.VMEM_SHARED((n_rows, 16), jnp.float32),
        pltpu.SemaphoreType.DMA,
        pltpu.SemaphoreType.DMA,
        pltpu.SemaphoreType.DMA,
        pltpu.SemaphoreType.DMA,
        pltpu.SemaphoreType.DMA,
    ]

    def body(table, src_hbm, dst_hbm, *rest):
        if with_deg:
            (agg_out, deg_out, src_v, dst_v, buf_a, buf_b, ones_v, z16,
             agg_sh, deg_sh, ga, gb, sa, sb, so) = rest
        else:
            (agg_out, src_v, dst_v, buf_a, buf_b, ones_v, z16,
             agg_sh, deg_sh, ga, gb, sa, sb, so) = rest
        c = lax.axis_index("c")
        s = lax.axis_index("s")
        base = s * rows_per_tile
        my_tab = table.at[c]

        pltpu.sync_copy(src_hbm.at[pl.ds(s * nch, nch)], src_v)
        pltpu.sync_copy(dst_hbm.at[pl.ds(s * nch, nch)], dst_v)

        zv = jnp.zeros((16,), jnp.float32)

        @pl.loop(0, K * CHUNK * (DH // 16))
        def _(t):
            i = t // (DH // 16)
            k = t % (DH // 16)
            buf_a[i, pl.ds(k * 16, 16)] = zv

        @pl.loop(0, CHUNK)
        def _(i):
            z16[i, pl.ds(0, 16)] = zv
            ones_v[i, pl.ds(0, 16)] = zv + 1.0

        nzc = rows_per_tile // (K * CHUNK)
        for r in range(nzc):
            pltpu.sync_copy(buf_a, agg_sh.at[pl.ds(base + r * K * CHUNK, K * CHUNK)])
        rem = rows_per_tile - nzc * K * CHUNK
        if rem:
            pltpu.sync_copy(buf_a.at[pl.ds(0, rem)],
                            agg_sh.at[pl.ds(base + nzc * K * CHUNK, rem)])
        if with_deg:
            for r in range(rows_per_tile // CHUNK):
                pltpu.sync_copy(z16, deg_sh.at[pl.ds(base + r * CHUNK, CHUNK)])
        plsc.subcore_barrier()

        def issue_g(g, buf, sem):
            for k in range(K):
                pltpu.async_copy(my_tab.at[src_v.at[g * K + k]],
                                 buf.at[pl.ds(k * CHUNK, CHUNK)], sem)

        def wait_g(buf, sem):
            pltpu.make_async_copy(my_tab.at[src_v.at[0]], buf, sem).wait()

        def issue_s(g, buf, sem, parity):
            for k in range(K):
                pltpu.sync_copy(buf.at[pl.ds(k * CHUNK, CHUNK)],
                                agg_sh.at[dst_v.at[g * K + k]], add=True)
            if with_deg:
                @pl.when(c == parity)
                def _():
                    for k in range(K):
                        pltpu.async_copy(ones_v, deg_sh.at[dst_v.at[g * K + k]],
                                         so, add=True)

        def wait_s(buf, sem):
            pass

        issue_g(0, buf_a, ga)
        issue_g(1, buf_b, gb)
        wait_g(buf_a, ga)
        issue_s(0, buf_a, sa, 0)

        @pl.loop(0, (T - 2) // 2)
        def _(jj):
            j1 = 2 * jj + 1
            wait_s(buf_a, sa)
            issue_g(j1 + 1, buf_a, ga)
            wait_g(buf_b, gb)
            issue_s(j1, buf_b, sb, 1)
            wait_s(buf_b, sb)
            issue_g(j1 + 2, buf_b, gb)
            wait_g(buf_a, ga)
            issue_s(j1 + 1, buf_a, sa, 0)

        wait_s(buf_a, sa)
        wait_g(buf_b, gb)
        issue_s(T - 1, buf_b, sb, 1)
        wait_s(buf_b, sb)
        if with_deg:
            @pl.loop(0, (T // 2) * K)
            def _(t):
                pltpu.make_async_copy(deg_out.at[c, pl.ds(0, CHUNK)], ones_v,
                                      so).wait()

        plsc.subcore_barrier()

        for r in range(nzc):
            rb = base + r * K * CHUNK
            pltpu.sync_copy(agg_sh.at[pl.ds(rb, K * CHUNK)], buf_a)
            pltpu.sync_copy(buf_a, agg_out.at[c, pl.ds(rb, K * CHUNK)])
        if rem:
            rb = base + nzc * K * CHUNK
            pltpu.sync_copy(agg_sh.at[pl.ds(rb, rem)], buf_a.at[pl.ds(0, rem)])
            pltpu.sync_copy(buf_a.at[pl.ds(0, rem)], agg_out.at[c, pl.ds(rb, rem)])
        if with_deg:
            for r in range(rows_per_tile // CHUNK):
                rb = base + r * CHUNK
                pltpu.sync_copy(deg_sh.at[pl.ds(rb, CHUNK)], z16)
                pltpu.sync_copy(z16, deg_out.at[c, pl.ds(rb, CHUNK)])

    return pl.kernel(
        body, out_type=out_type, mesh=mesh, scratch_types=scratch_types,
        compiler_params=pltpu.CompilerParams(use_tc_tiling_on_sc=False),
    )


def _sage_layer1(x2, aggp, deg, w_self, w_neigh, b):
    n_rows = x2.shape[1]
    blk = 1024

    def body(x_ref, a_ref, d_ref, ws_ref, wn_ref, b_ref, o_ref):
        x = jnp.concatenate([x_ref[0], x_ref[1]], axis=1)
        agg = jnp.concatenate([a_ref[0], a_ref[1]], axis=1)
        deg = d_ref[0, :, 0:1] + d_ref[1, :, 0:1]
        mean = agg / jnp.maximum(deg, 1.0)
        h = jnp.dot(x, ws_ref[...], preferred_element_type=jnp.float32)
        h = h + jnp.dot(mean, wn_ref[...], preferred_element_type=jnp.float32)
        h = h + b_ref[...]
        h = jnp.maximum(h, 0.0)
        o_ref[0] = h[:, :DH]
        o_ref[1] = h[:, DH:]

    return pl.pallas_call(
        body,
        grid=(n_rows // blk,),
        in_specs=[
            pl.BlockSpec((NC, blk, DH), lambda i: (0, i, 0)),
            pl.BlockSpec((NC, blk, DH), lambda i: (0, i, 0)),
            pl.BlockSpec((NC, blk, 16), lambda i: (0, i, 0)),
            pl.BlockSpec((D, D), lambda i: (0, 0)),
            pl.BlockSpec((D, D), lambda i: (0, 0)),
            pl.BlockSpec((1, D), lambda i: (0, 0)),
        ],
        out_specs=pl.BlockSpec((NC, blk, DH), lambda i: (0, i, 0)),
        out_shape=jax.ShapeDtypeStruct((NC, n_rows, DH), jnp.float32),
    )(x2, aggp, deg, w_self, w_neigh, b.reshape(1, D))


def _sage_layer2(h2, aggp, deg, x2, w_self, w_neigh, b, w_res, w_mlp_pad, b_mlp_pad):
    n_rows = h2.shape[1]
    blk = 1024

    def body(h_ref, a_ref, d_ref, x_ref, ws_ref, wn_ref, b_ref, wr_ref, wm_ref, bm_ref, o_ref):
        h1 = jnp.concatenate([h_ref[0], h_ref[1]], axis=1)
        x = jnp.concatenate([x_ref[0], x_ref[1]], axis=1)
        agg = jnp.concatenate([a_ref[0], a_ref[1]], axis=1)
        deg = d_ref[0, :, 0:1] + d_ref[1, :, 0:1]
        mean = agg / jnp.maximum(deg, 1.0)
        out = jnp.dot(h1, ws_ref[...], preferred_element_type=jnp.float32)
        out = out + jnp.dot(mean, wn_ref[...], preferred_element_type=jnp.float32)
        out = out + jnp.dot(x, wr_ref[...], preferred_element_type=jnp.float32)
        out = out + b_ref[...]
        o_ref[...] = jnp.dot(out, wm_ref[...], preferred_element_type=jnp.float32) + bm_ref[...]

    return pl.pallas_call(
        body,
        grid=(n_rows // blk,),
        in_specs=[
            pl.BlockSpec((NC, blk, DH), lambda i: (0, i, 0)),
            pl.BlockSpec((NC, blk, DH), lambda i: (0, i, 0)),
            pl.BlockSpec((NC, blk, 16), lambda i: (0, i, 0)),
            pl.BlockSpec((NC, blk, DH), lambda i: (0, i, 0)),
            pl.BlockSpec((D, D), lambda i: (0, 0)),
            pl.BlockSpec((D, D), lambda i: (0, 0)),
            pl.BlockSpec((1, D), lambda i: (0, 0)),
            pl.BlockSpec((D, D), lambda i: (0, 0)),
            pl.BlockSpec((D, D), lambda i: (0, 0)),
            pl.BlockSpec((1, D), lambda i: (0, 0)),
        ],
        out_specs=pl.BlockSpec((blk, D), lambda i: (i, 0)),
        out_shape=jax.ShapeDtypeStruct((n_rows, D), jnp.float32),
    )(h2, aggp, deg, x2, w_self, w_neigh, b.reshape(1, D), w_res, w_mlp_pad, b_mlp_pad)


def kernel(node_feats, edge_index, W_self1, W_neigh1, b1, W_self2, W_neigh2, b2, W_res, W_mlp, b_mlp):
    n = node_feats.shape[0]
    e = edge_index.shape[1]
    src = edge_index[0]
    dst = edge_index[1]

    nch = -(-e // (NS * CHUNK))
    nch = -(-nch // 8) * 8
    e_pad = NS * nch * CHUNK
    src_p = jnp.concatenate([src, jnp.zeros((e_pad - e,), jnp.int32)])
    dst_p = jnp.concatenate([dst, jnp.full((e_pad - e,), n, jnp.int32)])
    src_p = src_p.reshape(e_pad // CHUNK, CHUNK)
    dst_p = dst_p.reshape(e_pad // CHUNK, CHUNK)

    rows_per_tile = -(-(n + 1) // (NS * CHUNK)) * CHUNK
    n_rows = rows_per_tile * NS

    x_pad = jnp.zeros((n_rows, D), jnp.float32).at[:n].set(node_feats)
    x2 = jnp.stack([x_pad[:, :DH], x_pad[:, DH:]])

    seg1 = _make_seg_kernel(nch, n_rows, rows_per_tile, with_deg=True)
    aggp1, deg = seg1(x2, src_p, dst_p)

    h2 = _sage_layer1(x2, aggp1, deg, W_self1, W_neigh1, b1)

    seg2 = _make_seg_kernel(nch, n_rows, rows_per_tile, with_deg=False)
    (aggp2,) = seg2(h2, src_p, dst_p)

    w_mlp_pad = jnp.zeros((D, D), jnp.float32).at[:, : W_mlp.shape[1]].set(W_mlp)
    b_mlp_pad = jnp.zeros((1, D), jnp.float32).at[0, : W_mlp.shape[1]].set(b_mlp)

    out = _sage_layer2(h2, aggp2, deg, x2, W_self2, W_neigh2, b2, W_res, w_mlp_pad, b_mlp_pad)
    return out[:n, : W_mlp.shape[1]]
